# baseline (speedup 1.0000x reference)
import jax
import jax.numpy as jnp
from jax.experimental import pallas as pl

_N = 10000
_E = 320000
_NP = 10240
_BM = 512
_D = 64


def _mlp_kernel(x_ref, w_ref, b_ref, o_ref):
    o_ref[...] = (
        jnp.dot(x_ref[...], w_ref[...], preferred_element_type=jnp.float32)
        + b_ref[...]
    )


def _mlp(x, w, b):
    m, k = x.shape
    d = w.shape[1]
    return pl.pallas_call(
        _mlp_kernel,
        grid=(m // _BM,),
        in_specs=[
            pl.BlockSpec((_BM, k), lambda i: (i, 0)),
            pl.BlockSpec((k, d), lambda i: (0, 0)),
            pl.BlockSpec((1, d), lambda i: (0, 0)),
        ],
        out_specs=pl.BlockSpec((_BM, d), lambda i: (i, 0)),
        out_shape=jax.ShapeDtypeStruct((m, d), jnp.float32),
    )(x, w, b.reshape(1, d))


def _prop_kernel(a_ref, z_ref, i_ref, o_ref):
    agg = jnp.dot(
        a_ref[...],
        z_ref[...].astype(jnp.bfloat16),
        preferred_element_type=jnp.float32,
    )
    o_ref[...] = 0.5 * i_ref[...] + 0.5 * agg


def _propagate(a, z, init):
    return pl.pallas_call(
        _prop_kernel,
        grid=(_NP // _BM,),
        in_specs=[
            pl.BlockSpec((_BM, _NP), lambda i: (i, 0)),
            pl.BlockSpec((_NP, _D), lambda i: (0, 0)),
            pl.BlockSpec((_BM, _D), lambda i: (i, 0)),
        ],
        out_specs=pl.BlockSpec((_BM, _D), lambda i: (i, 0)),
        out_shape=jax.ShapeDtypeStruct((_NP, _D), jnp.float32),
    )(a, z, init)


def kernel(features, edge_index, W1, b1, W2, b2):
    src = edge_index[0]
    dst = edge_index[1]
    deg = jnp.zeros((_N,), jnp.float32).at[dst].add(1.0)
    dis = jnp.where(deg > 0, jax.lax.rsqrt(jnp.clip(deg, 1.0)), 0.0)
    w = dis[dst] * dis[src]
    a = (
        jnp.zeros((_NP, _NP), jnp.bfloat16)
        .at[dst, src]
        .add(w.astype(jnp.bfloat16))
    )

    xp = jnp.pad(features, ((0, _NP - _N), (0, 0)))
    init1 = _mlp(xp, W1, b1)
    z = init1
    for _ in range(8):
        z = _propagate(a, z, init1)

    n_cls = W2.shape[1]
    w2p = jnp.pad(W2, ((0, 0), (0, _D - n_cls)))
    b2p = jnp.pad(b2, (0, _D - n_cls))
    init2 = _mlp(z, w2p, b2p)
    z2 = init2
    for _ in range(8):
        z2 = _propagate(a, z2, init2)

    return z2[:_N, :n_cls]


# speedup vs baseline: 1.2396x; 1.2396x over previous
import jax
import jax.numpy as jnp
from jax.experimental import pallas as pl

_N = 10000
_E = 320000
_NP = 10240
_BM = 512
_D = 64


def _mlp_kernel(x_ref, w_ref, b_ref, o_ref):
    o_ref[...] = (
        jnp.dot(x_ref[...], w_ref[...], preferred_element_type=jnp.float32)
        + b_ref[...]
    )


def _mlp(x, w, b):
    m, k = x.shape
    d = w.shape[1]
    return pl.pallas_call(
        _mlp_kernel,
        grid=(m // _BM,),
        in_specs=[
            pl.BlockSpec((_BM, k), lambda i: (i, 0)),
            pl.BlockSpec((k, d), lambda i: (0, 0)),
            pl.BlockSpec((1, d), lambda i: (0, 0)),
        ],
        out_specs=pl.BlockSpec((_BM, d), lambda i: (i, 0)),
        out_shape=jax.ShapeDtypeStruct((m, d), jnp.float32),
    )(x, w, b.reshape(1, d))


def _prop_kernel(a_ref, z_ref, i_ref, o_ref):
    agg = jnp.dot(
        a_ref[...],
        z_ref[...].astype(jnp.bfloat16),
        preferred_element_type=jnp.float32,
    )
    o_ref[...] = 0.5 * i_ref[...] + 0.5 * agg


def _propagate(a, z, init):
    return pl.pallas_call(
        _prop_kernel,
        grid=(_NP // _BM,),
        in_specs=[
            pl.BlockSpec((_BM, _NP), lambda i: (i, 0)),
            pl.BlockSpec((_NP, _D), lambda i: (0, 0)),
            pl.BlockSpec((_BM, _D), lambda i: (i, 0)),
        ],
        out_specs=pl.BlockSpec((_BM, _D), lambda i: (i, 0)),
        out_shape=jax.ShapeDtypeStruct((_NP, _D), jnp.float32),
    )(a, z, init)


def kernel(features, edge_index, W1, b1, W2, b2):
    src = edge_index[0]
    dst = edge_index[1]
    deg = jnp.zeros((_N,), jnp.float32).at[dst].add(1.0)
    dis = jnp.where(deg > 0, jax.lax.rsqrt(jnp.clip(deg, 1.0)), 0.0)
    w = dis[dst] * dis[src]
    a = (
        jnp.zeros((_NP, _NP), jnp.float32)
        .at[dst, src]
        .add(w)
        .astype(jnp.bfloat16)
    )

    xp = jnp.pad(features, ((0, _NP - _N), (0, 0)))
    init1 = _mlp(xp, W1, b1)
    z = init1
    for _ in range(8):
        z = _propagate(a, z, init1)

    n_cls = W2.shape[1]
    w2p = jnp.pad(W2, ((0, 0), (0, _D - n_cls)))
    b2p = jnp.pad(b2, (0, _D - n_cls))
    init2 = _mlp(z, w2p, b2p)
    z2 = init2
    for _ in range(8):
        z2 = _propagate(a, z2, init2)

    return z2[:_N, :n_cls]


# speedup vs baseline: 1.2657x; 1.0211x over previous
import jax
import jax.numpy as jnp
from jax.experimental import pallas as pl
from jax.experimental.pallas import tpu as pltpu

_N = 10000
_E = 320000
_NP = 10240
_BM = 512
_D = 64


def _mlp_kernel(x_ref, w_ref, b_ref, o_ref):
    o_ref[...] = (
        jnp.dot(x_ref[...], w_ref[...], preferred_element_type=jnp.float32)
        + b_ref[...]
    )


def _mlp(x, w, b):
    m, k = x.shape
    d = w.shape[1]
    return pl.pallas_call(
        _mlp_kernel,
        grid=(m // _BM,),
        in_specs=[
            pl.BlockSpec((_BM, k), lambda i: (i, 0)),
            pl.BlockSpec((k, d), lambda i: (0, 0)),
            pl.BlockSpec((1, d), lambda i: (0, 0)),
        ],
        out_specs=pl.BlockSpec((_BM, d), lambda i: (i, 0)),
        out_shape=jax.ShapeDtypeStruct((m, d), jnp.float32),
    )(x, w, b.reshape(1, d))


_K = 8


def _block_kernel(a_ref, init_ref, o_ref, za_ref, zb_ref):
    t = pl.program_id(0)
    i = pl.program_id(1)
    base = i * _BM
    zprev = jnp.where((t % 2) == 1, za_ref[...], zb_ref[...])
    zprev = jnp.where(t == 0, init_ref[...], zprev)
    agg = jnp.dot(
        a_ref[...],
        zprev.astype(jnp.bfloat16),
        preferred_element_type=jnp.float32,
    )
    newz = 0.5 * init_ref[pl.ds(base, _BM), :] + 0.5 * agg

    @pl.when((t % 2) == 0)
    def _():
        za_ref[pl.ds(base, _BM), :] = newz

    @pl.when((t % 2) == 1)
    def _():
        zb_ref[pl.ds(base, _BM), :] = newz

    o_ref[...] = newz


def _vgcn_block(a, init):
    return pl.pallas_call(
        _block_kernel,
        grid=(_K, _NP // _BM),
        in_specs=[
            pl.BlockSpec((_BM, _NP), lambda t, i: (i, 0)),
            pl.BlockSpec((_NP, _D), lambda t, i: (0, 0)),
        ],
        out_specs=pl.BlockSpec((_BM, _D), lambda t, i: (i, 0)),
        out_shape=jax.ShapeDtypeStruct((_NP, _D), jnp.float32),
        scratch_shapes=[
            pltpu.VMEM((_NP, _D), jnp.float32),
            pltpu.VMEM((_NP, _D), jnp.float32),
        ],
    )(a, init)


def kernel(features, edge_index, W1, b1, W2, b2):
    src = edge_index[0]
    dst = edge_index[1]
    deg = jnp.zeros((_N,), jnp.float32).at[dst].add(1.0)
    dis = jnp.where(deg > 0, jax.lax.rsqrt(jnp.clip(deg, 1.0)), 0.0)
    w = dis[dst] * dis[src]
    a = (
        jnp.zeros((_NP, _NP), jnp.float32)
        .at[dst, src]
        .add(w)
        .astype(jnp.bfloat16)
    )

    xp = jnp.pad(features, ((0, _NP - _N), (0, 0)))
    init1 = _mlp(xp, W1, b1)
    z = _vgcn_block(a, init1)

    n_cls = W2.shape[1]
    w2p = jnp.pad(W2, ((0, 0), (0, _D - n_cls)))
    b2p = jnp.pad(b2, (0, _D - n_cls))
    init2 = _mlp(z, w2p, b2p)
    z2 = _vgcn_block(a, init2)

    return z2[:_N, :n_cls]
